# Initial kernel scaffold; baseline (speedup 1.0000x reference)
import jax
import jax.numpy as jnp
from jax.experimental import pallas as pl


def kernel(instances, table):
    raise NotImplementedError("write your pallas kernel here")


# speedup vs baseline: 13.9145x; 13.9145x over previous
import functools

import jax
import jax.numpy as jnp
from jax import lax
from jax.experimental import pallas as pl
from jax.experimental.pallas import tpu as pltpu
from jax.experimental.pallas import tpu_sc as plsc

_NUM_CORES = 2
_NUM_SUBCORES = 16
_NW = _NUM_CORES * _NUM_SUBCORES
_LANES = 16

_CHUNKS = ((0, 128), (128, 72))


def _avg_pool_sc(idx_flat, table, batch, hist, dim):
    spw = batch // _NW
    sblk = 64
    nblk = spw // sblk
    inv_l = float(1.0 / hist)

    mesh = plsc.VectorSubcoreMesh(core_axis_name="c", subcore_axis_name="s")

    @functools.partial(
        pl.kernel,
        mesh=mesh,
        compiler_params=pltpu.CompilerParams(use_tc_tiling_on_sc=False),
        out_type=jax.ShapeDtypeStruct((batch, dim), jnp.float32),
        scratch_types=[
            pltpu.VMEM((sblk * hist,), jnp.int32),
            pltpu.VMEM((hist, dim), jnp.float32),
            pltpu.VMEM((hist, dim), jnp.float32),
            pltpu.VMEM((spw, dim), jnp.float32),
            pltpu.SemaphoreType.DMA,
            pltpu.SemaphoreType.DMA,
        ],
    )
    def k(idx_hbm, table_hbm, out_hbm, idx_v, rows_a, rows_b, out_v,
          sem_a, sem_b):
        wid = lax.axis_index("s") * _NUM_CORES + lax.axis_index("c")
        base = wid * spw

        def gather_start(rows, sem, s_in_blk):
            off = s_in_blk * hist
            for c0, cn in _CHUNKS:
                pltpu.make_async_copy(
                    table_hbm.at[idx_v.at[pl.ds(off + c0, cn)]],
                    rows.at[pl.ds(c0, cn), :],
                    sem,
                ).start()

        def gather_wait(rows, sem):
            for c0, cn in _CHUNKS:
                pltpu.make_async_copy(
                    table_hbm.at[idx_v.at[pl.ds(c0, cn)]],
                    rows.at[pl.ds(c0, cn), :],
                    sem,
                ).wait()

        def reduce_sample(rows, s_out):
            zero = jnp.zeros((_LANES,), jnp.float32)

            def body(i, accs):
                lo, hi = accs
                r = i * 8
                for h0 in (0, _LANES):
                    t = [rows[r + j, pl.ds(h0, _LANES)] for j in range(8)]
                    s01, s23 = t[0] + t[1], t[2] + t[3]
                    s45, s67 = t[4] + t[5], t[6] + t[7]
                    s = (s01 + s23) + (s45 + s67)
                    if h0 == 0:
                        lo = lo + s
                    else:
                        hi = hi + s
                return lo, hi

            lo, hi = lax.fori_loop(0, hist // 8, body, (zero, zero))
            out_v[s_out, pl.ds(0, _LANES)] = lo * inv_l
            out_v[s_out, pl.ds(_LANES, _LANES)] = hi * inv_l

        @pl.loop(0, nblk)
        def _blk(blk):
            blk_sample = base + blk * sblk
            pltpu.sync_copy(
                idx_hbm.at[pl.ds(blk_sample * hist, sblk * hist)], idx_v)
            gather_start(rows_a, sem_a, 0)

            @pl.loop(0, sblk, step=2)
            def _s(s2):
                gather_start(rows_b, sem_b, s2 + 1)
                gather_wait(rows_a, sem_a)
                reduce_sample(rows_a, blk * sblk + s2)

                @pl.when(s2 + 2 < sblk)
                def _():
                    gather_start(rows_a, sem_a, s2 + 2)

                gather_wait(rows_b, sem_b)
                reduce_sample(rows_b, blk * sblk + s2 + 1)

        pltpu.sync_copy(out_v, out_hbm.at[pl.ds(base, spw), :])

    return k(idx_flat, table)


def kernel(instances, table):
    batch, hist = instances.shape
    _, dim = table.shape
    idx_flat = instances.astype(jnp.int32).reshape(batch * hist)
    return _avg_pool_sc(idx_flat, table, batch, hist, dim)


# speedup vs baseline: 15.6073x; 1.1217x over previous
import functools

import jax
import jax.numpy as jnp
from jax import lax
from jax.experimental import pallas as pl
from jax.experimental.pallas import tpu as pltpu
from jax.experimental.pallas import tpu_sc as plsc

_NUM_CORES = 2
_NUM_SUBCORES = 16
_NW = _NUM_CORES * _NUM_SUBCORES
_LANES = 16

_CHUNKS = ((0, 200),)
_NBUF = 4


def _avg_pool_sc(idx_flat, table, batch, hist, dim):
    spw = batch // _NW
    sblk = 128
    nblk = spw // sblk
    inv_l = float(1.0 / hist)

    mesh = plsc.VectorSubcoreMesh(core_axis_name="c", subcore_axis_name="s")

    @functools.partial(
        pl.kernel,
        mesh=mesh,
        compiler_params=pltpu.CompilerParams(use_tc_tiling_on_sc=False),
        out_type=jax.ShapeDtypeStruct((batch, dim), jnp.float32),
        scratch_types=(
            [pltpu.VMEM((sblk * hist,), jnp.int32)]
            + [pltpu.VMEM((hist, dim), jnp.float32)
               for _ in range(_NBUF)]
            + [pltpu.VMEM((spw, dim), jnp.float32)]
            + [pltpu.SemaphoreType.DMA for _ in range(_NBUF)]
        ),
    )
    def k(idx_hbm, table_hbm, out_hbm, idx_v, *rest):
        rows = rest[:_NBUF]
        out_v = rest[_NBUF]
        sems = rest[_NBUF + 1:]
        wid = lax.axis_index("s") * _NUM_CORES + lax.axis_index("c")
        base = wid * spw

        def gather_start(rows, sem, s_in_blk):
            off = s_in_blk * hist
            for c0, cn in _CHUNKS:
                pltpu.make_async_copy(
                    table_hbm.at[idx_v.at[pl.ds(off + c0, cn)]],
                    rows.at[pl.ds(c0, cn), :],
                    sem,
                ).start()

        def gather_wait(rows, sem):
            for c0, cn in _CHUNKS:
                pltpu.make_async_copy(
                    table_hbm.at[idx_v.at[pl.ds(c0, cn)]],
                    rows.at[pl.ds(c0, cn), :],
                    sem,
                ).wait()

        def reduce_sample(rows, s_out):
            zero = jnp.zeros((_LANES,), jnp.float32)

            def body(i, accs):
                lo, hi = accs
                r = i * 8
                for h0 in (0, _LANES):
                    t = [rows[r + j, pl.ds(h0, _LANES)] for j in range(8)]
                    s01, s23 = t[0] + t[1], t[2] + t[3]
                    s45, s67 = t[4] + t[5], t[6] + t[7]
                    s = (s01 + s23) + (s45 + s67)
                    if h0 == 0:
                        lo = lo + s
                    else:
                        hi = hi + s
                return lo, hi

            lo, hi = lax.fori_loop(0, hist // 8, body, (zero, zero))
            out_v[s_out, pl.ds(0, _LANES)] = lo * inv_l
            out_v[s_out, pl.ds(_LANES, _LANES)] = hi * inv_l

        @pl.loop(0, nblk)
        def _blk(blk):
            blk_sample = base + blk * sblk
            pltpu.sync_copy(
                idx_hbm.at[pl.ds(blk_sample * hist, sblk * hist)], idx_v)
            for j in range(_NBUF - 1):
                gather_start(rows[j], sems[j], j)

            @pl.loop(0, sblk, step=_NBUF)
            def _s(s0):
                for j in range(_NBUF):
                    s = s0 + j
                    gather_wait(rows[j], sems[j])
                    reduce_sample(rows[j], blk * sblk + s)
                    jn = (j + _NBUF - 1) % _NBUF

                    @pl.when(s + _NBUF - 1 < sblk)
                    def _():
                        gather_start(rows[jn], sems[jn], s + _NBUF - 1)

        pltpu.sync_copy(out_v, out_hbm.at[pl.ds(base, spw), :])

    return k(idx_flat, table)


def kernel(instances, table):
    batch, hist = instances.shape
    _, dim = table.shape
    idx_flat = instances.astype(jnp.int32).reshape(batch * hist)
    return _avg_pool_sc(idx_flat, table, batch, hist, dim)
